# baseline (speedup 1.0000x reference)
import functools

import jax
import jax.numpy as jnp
from jax import lax
from jax.experimental import pallas as pl
from jax.experimental.pallas import tpu as pltpu
from jax.experimental.pallas import tpu_sc as plsc

K = 8
D = 1024
BLK = 512
NC = 2
NS = 16
NW = NC * NS
L = 16

_SC_MESH = plsc.VectorSubcoreMesh(core_axis_name="c", subcore_axis_name="s")
_SC_PARAMS = pltpu.CompilerParams(needs_layout_passes=False)


def _wid():
    return lax.axis_index("s") * NC + lax.axis_index("c")


def _sc_counts_body(cls_hbm, counts_hbm, cls_v, cnt_v):
    wid = _wid()
    chunk = cls_hbm.shape[0] // NW
    pltpu.sync_copy(cls_hbm.at[pl.ds(wid * chunk, chunk)], cls_v)
    lane = lax.iota(jnp.int32, L)
    counts = jnp.zeros((L,), jnp.int32)
    for v in range(chunk // L):
        c = cls_v[pl.ds(v * L, L)]
        for k in range(K):
            pc = jnp.sum((c == k).astype(jnp.int32))
            counts = counts + jnp.where(lane == k, pc, 0)
    cnt_v[...] = counts
    pltpu.sync_copy(cnt_v, counts_hbm.at[wid])


def _sc_counts(cls):
    n = cls.shape[0]
    chunk = n // NW
    f = pl.kernel(
        _sc_counts_body,
        out_type=jax.ShapeDtypeStruct((NW, L), jnp.int32),
        mesh=_SC_MESH,
        scratch_types=[
            pltpu.VMEM((chunk,), jnp.int32),
            pltpu.VMEM((L,), jnp.int32),
        ],
        compiler_params=_SC_PARAMS,
    )
    return f(cls)


def _sc_route_body(cls_hbm, base_hbm, feats_hbm, xsorted_hbm, dest_hbm,
                   cls_v, base_v, run_v, dest_v, rows_v, sem):
    wid = _wid()
    chunk = cls_hbm.shape[0] // NW
    half = chunk // 2
    pltpu.sync_copy(cls_hbm.at[pl.ds(wid * chunk, chunk)], cls_v)
    pltpu.sync_copy(base_hbm.at[wid], base_v)
    run_v[...] = jnp.zeros((L,), jnp.int32)
    lane = lax.iota(jnp.int32, L)
    for v in range(chunk // L):
        c = cls_v[pl.ds(v * L, L)]
        bg = plsc.load_gather(base_v, [c])
        rg = plsc.load_gather(run_v, [c])
        within = jnp.zeros((L,), jnp.int32)
        newc = jnp.zeros((L,), jnp.int32)
        for k in range(K):
            eq = c == k
            cs = plsc.cumsum(eq.astype(jnp.int32))
            within = jnp.where(eq, cs - 1, within)
            pc = cs[L - 1]
            newc = newc + jnp.where(lane == k, pc, 0)
        dest_v[v * L // half, pl.ds((v * L) % half, L)] = bg + rg + within
        run_v[...] = run_v[...] + newc
    pltpu.sync_copy(dest_v, dest_hbm.at[wid])
    for h in range(2):
        pltpu.sync_copy(feats_hbm.at[pl.ds(wid * chunk + h * half, half)], rows_v)
        pltpu.async_copy(rows_v, xsorted_hbm.at[dest_v.at[h]], sem).wait()


def _sc_route(cls, base, feats):
    n = cls.shape[0]
    chunk = n // NW
    half = chunk // 2
    f = pl.kernel(
        _sc_route_body,
        out_type=(
            jax.ShapeDtypeStruct((n, D), jnp.float32),
            jax.ShapeDtypeStruct((NW, 2, half), jnp.int32),
        ),
        mesh=_SC_MESH,
        scratch_types=[
            pltpu.VMEM((chunk,), jnp.int32),
            pltpu.VMEM((L,), jnp.int32),
            pltpu.VMEM((L,), jnp.int32),
            pltpu.VMEM((2, half), jnp.int32),
            pltpu.VMEM((half, D), jnp.float32),
            pltpu.SemaphoreType.DMA,
        ],
        compiler_params=_SC_PARAMS,
    )
    return f(cls, base, feats)


def _sc_unsort_body(ysorted_hbm, dest_hbm, y_hbm, dest_v, rows_v, sem):
    wid = _wid()
    half = dest_hbm.shape[2]
    chunk = 2 * half
    pltpu.sync_copy(dest_hbm.at[wid], dest_v)
    for h in range(2):
        pltpu.async_copy(ysorted_hbm.at[dest_v.at[h]], rows_v, sem).wait()
        pltpu.sync_copy(rows_v, y_hbm.at[pl.ds(wid * chunk + h * half, half)])


def _sc_unsort(ysorted, dest):
    n = ysorted.shape[0]
    half = dest.shape[2]
    f = pl.kernel(
        _sc_unsort_body,
        out_type=jax.ShapeDtypeStruct((n, D), jnp.float32),
        mesh=_SC_MESH,
        scratch_types=[
            pltpu.VMEM((2, half), jnp.int32),
            pltpu.VMEM((half, D), jnp.float32),
            pltpu.SemaphoreType.DMA,
        ],
        compiler_params=_SC_PARAMS,
    )
    return f(ysorted, dest)


def _gmm_body(bid_ref, cid_ref, gs_ref, ge_ref, x_ref, w_ref, b_ref, o_ref):
    g = pl.program_id(0)
    start = gs_ref[g]
    end = ge_ref[g]
    rows = bid_ref[g] * BLK + lax.broadcasted_iota(jnp.int32, (BLK, 1), 0)
    m = (rows >= start) & (rows < end)
    xm = jnp.where(m, x_ref[...], 0.0).astype(jnp.bfloat16)
    y = lax.dot_general(xm, w_ref[0], (((1,), (1,)), ((), ())),
                        preferred_element_type=jnp.float32)
    y = y + jnp.where(m, b_ref[0], 0.0)
    prev = bid_ref[jnp.maximum(g - 1, 0)]
    first = (g == 0) | (bid_ref[g] != prev)

    @pl.when(first)
    def _():
        o_ref[...] = y

    @pl.when(jnp.logical_not(first))
    def _():
        o_ref[...] = o_ref[...] + y


def _pair_metadata(counts, nb):
    g_total = nb + K - 1
    ends = jnp.cumsum(counts)
    starts = ends - counts
    nonempty = counts > 0
    fb = starts // BLK
    lb = jnp.where(nonempty, (ends - 1) // BLK, 0)
    nbk = jnp.where(nonempty, lb - fb + 1, 0)
    pair_cls = jnp.repeat(jnp.arange(K, dtype=jnp.int32), nbk,
                          total_repeat_length=g_total)
    first_pair = jnp.cumsum(nbk) - nbk
    g_idx = jnp.arange(g_total, dtype=jnp.int32)
    valid = g_idx < jnp.sum(nbk)
    pair_blk = fb[pair_cls] + g_idx - first_pair[pair_cls]
    pair_blk = jnp.where(valid, pair_blk, nb - 1)
    gs = jnp.where(valid, starts[pair_cls], 0)
    ge = jnp.where(valid, ends[pair_cls], 0)
    return (pair_blk.astype(jnp.int32), pair_cls.astype(jnp.int32),
            gs.astype(jnp.int32), ge.astype(jnp.int32))


def _grouped_matmul(x_sorted, Ws, bs, pair_blk, pair_cls, gs, ge, interpret=False):
    n = x_sorted.shape[0]
    nb = n // BLK
    g_total = nb + K - 1
    grid_spec = pltpu.PrefetchScalarGridSpec(
        num_scalar_prefetch=4,
        grid=(g_total,),
        in_specs=[
            pl.BlockSpec((BLK, D), lambda g, bid, cid, s, e: (bid[g], 0)),
            pl.BlockSpec((1, D, D), lambda g, bid, cid, s, e: (cid[g], 0, 0)),
            pl.BlockSpec((1, 1, D), lambda g, bid, cid, s, e: (cid[g], 0, 0)),
        ],
        out_specs=pl.BlockSpec((BLK, D), lambda g, bid, cid, s, e: (bid[g], 0)),
    )
    return pl.pallas_call(
        _gmm_body,
        grid_spec=grid_spec,
        out_shape=jax.ShapeDtypeStruct((n, D), jnp.float32),
        compiler_params=pltpu.CompilerParams(
            dimension_semantics=("arbitrary",)),
        interpret=interpret,
    )(pair_blk, pair_cls, gs, ge, x_sorted, Ws.astype(jnp.bfloat16),
      bs.reshape(K, 1, D))


def kernel(x, Ws, bs):
    n = x.shape[0]
    cls = x[:, 0].astype(jnp.int32)
    feats = x[:, 1:]
    counts_all = _sc_counts(cls)
    totals = jnp.sum(counts_all, axis=0)
    class_start = jnp.cumsum(totals) - totals
    prefix_w = jnp.cumsum(counts_all, axis=0) - counts_all
    base = (class_start[None, :] + prefix_w).astype(jnp.int32)
    x_sorted, dest = _sc_route(cls, base, feats)
    pair_blk, pair_cls, gs, ge = _pair_metadata(totals[:K], n // BLK)
    y_sorted = _grouped_matmul(x_sorted, Ws, bs, pair_blk, pair_cls, gs, ge)
    return _sc_unsort(y_sorted, dest)


# speedup vs baseline: 1.1250x; 1.1250x over previous
import functools

import jax
import jax.numpy as jnp
from jax import lax
from jax.experimental import pallas as pl
from jax.experimental.pallas import tpu as pltpu
from jax.experimental.pallas import tpu_sc as plsc

K = 8
D = 1024
BLK = 512
NC = 2
NS = 16
NW = NC * NS
L = 16

_SC_MESH = plsc.VectorSubcoreMesh(core_axis_name="c", subcore_axis_name="s")
_SC_PARAMS = pltpu.CompilerParams(needs_layout_passes=False)


def _wid():
    return lax.axis_index("s") * NC + lax.axis_index("c")


def _sc_counts_body(cls_hbm, counts_hbm, cls_v, cnt_v):
    wid = _wid()
    chunk = cls_hbm.shape[0] // NW
    pltpu.sync_copy(cls_hbm.at[pl.ds(wid * chunk, chunk)], cls_v)
    lane = lax.iota(jnp.int32, L)
    counts = jnp.zeros((L,), jnp.int32)
    for v in range(chunk // L):
        c = cls_v[pl.ds(v * L, L)]
        for k in range(K):
            pc = jnp.sum((c == k).astype(jnp.int32))
            counts = counts + jnp.where(lane == k, pc, 0)
    cnt_v[...] = counts
    pltpu.sync_copy(cnt_v, counts_hbm.at[wid])


def _sc_counts(cls):
    n = cls.shape[0]
    chunk = n // NW
    f = pl.kernel(
        _sc_counts_body,
        out_type=jax.ShapeDtypeStruct((NW, L), jnp.int32),
        mesh=_SC_MESH,
        scratch_types=[
            pltpu.VMEM((chunk,), jnp.int32),
            pltpu.VMEM((L,), jnp.int32),
        ],
        compiler_params=_SC_PARAMS,
    )
    return f(cls)


def _sc_route_body(cls_hbm, base_hbm, feats_hbm, xsorted_hbm, dest_hbm,
                   cls_v, base_v, run_v, dest_v, rows_v, sem):
    wid = _wid()
    chunk = cls_hbm.shape[0] // NW
    half = chunk // 2
    pltpu.sync_copy(cls_hbm.at[pl.ds(wid * chunk, chunk)], cls_v)
    pltpu.sync_copy(base_hbm.at[wid], base_v)
    run_v[...] = jnp.zeros((L,), jnp.int32)
    lane = lax.iota(jnp.int32, L)
    for v in range(chunk // L):
        c = cls_v[pl.ds(v * L, L)]
        bg = plsc.load_gather(base_v, [c])
        rg = plsc.load_gather(run_v, [c])
        within = jnp.zeros((L,), jnp.int32)
        newc = jnp.zeros((L,), jnp.int32)
        for k in range(K):
            eq = c == k
            cs = plsc.cumsum(eq.astype(jnp.int32))
            within = jnp.where(eq, cs - 1, within)
            pc = cs[L - 1]
            newc = newc + jnp.where(lane == k, pc, 0)
        dest_v[v * L // half, pl.ds((v * L) % half, L)] = bg + rg + within
        run_v[...] = run_v[...] + newc
    pltpu.sync_copy(dest_v, dest_hbm.at[wid])
    for h in range(2):
        pltpu.sync_copy(feats_hbm.at[pl.ds(wid * chunk + h * half, half)], rows_v)
        pltpu.async_copy(rows_v, xsorted_hbm.at[dest_v.at[h]], sem).wait()


def _sc_route(cls, base, feats):
    n = cls.shape[0]
    chunk = n // NW
    half = chunk // 2
    f = pl.kernel(
        _sc_route_body,
        out_type=(
            jax.ShapeDtypeStruct((n, D), jnp.float32),
            jax.ShapeDtypeStruct((NW, 2, half), jnp.int32),
        ),
        mesh=_SC_MESH,
        scratch_types=[
            pltpu.VMEM((chunk,), jnp.int32),
            pltpu.VMEM((L,), jnp.int32),
            pltpu.VMEM((L,), jnp.int32),
            pltpu.VMEM((2, half), jnp.int32),
            pltpu.VMEM((half, D), jnp.float32),
            pltpu.SemaphoreType.DMA,
        ],
        compiler_params=_SC_PARAMS,
    )
    return f(cls, base, feats)


def _sc_unsort_body(ysorted_hbm, dest_hbm, y_hbm, dest_v, rows_v, sem):
    wid = _wid()
    half = dest_hbm.shape[2]
    chunk = 2 * half
    pltpu.sync_copy(dest_hbm.at[wid], dest_v)
    for h in range(2):
        pltpu.async_copy(ysorted_hbm.at[dest_v.at[h]], rows_v, sem).wait()
        pltpu.sync_copy(rows_v, y_hbm.at[pl.ds(wid * chunk + h * half, half)])


def _sc_unsort(ysorted, dest):
    n = ysorted.shape[0]
    half = dest.shape[2]
    f = pl.kernel(
        _sc_unsort_body,
        out_type=jax.ShapeDtypeStruct((n, D), jnp.float32),
        mesh=_SC_MESH,
        scratch_types=[
            pltpu.VMEM((2, half), jnp.int32),
            pltpu.VMEM((half, D), jnp.float32),
            pltpu.SemaphoreType.DMA,
        ],
        compiler_params=_SC_PARAMS,
    )
    return f(ysorted, dest)


def _gmm_body(bid_ref, cid_ref, gs_ref, ge_ref, x_ref, w_ref, b_ref, o_ref):
    g = pl.program_id(0)
    start = gs_ref[g]
    end = ge_ref[g]
    rows = bid_ref[g] * BLK + lax.broadcasted_iota(jnp.int32, (BLK, 1), 0)
    m = (rows >= start) & (rows < end)
    xm = jnp.where(m, x_ref[...], 0.0).astype(jnp.bfloat16)
    y = lax.dot_general(xm, w_ref[0].astype(jnp.bfloat16),
                        (((1,), (1,)), ((), ())),
                        preferred_element_type=jnp.float32)
    y = y + jnp.where(m, b_ref[0], 0.0)
    prev = bid_ref[jnp.maximum(g - 1, 0)]
    first = (g == 0) | (bid_ref[g] != prev)

    @pl.when(first)
    def _():
        o_ref[...] = y

    @pl.when(jnp.logical_not(first))
    def _():
        o_ref[...] = o_ref[...] + y


def _pair_metadata(counts, nb):
    g_total = nb + K - 1
    ends = jnp.cumsum(counts)
    starts = ends - counts
    nonempty = counts > 0
    fb = starts // BLK
    lb = jnp.where(nonempty, (ends - 1) // BLK, 0)
    nbk = jnp.where(nonempty, lb - fb + 1, 0)
    pair_cls = jnp.repeat(jnp.arange(K, dtype=jnp.int32), nbk,
                          total_repeat_length=g_total)
    first_pair = jnp.cumsum(nbk) - nbk
    g_idx = jnp.arange(g_total, dtype=jnp.int32)
    valid = g_idx < jnp.sum(nbk)
    pair_blk = fb[pair_cls] + g_idx - first_pair[pair_cls]
    pair_blk = jnp.where(valid, pair_blk, nb - 1)
    gs = jnp.where(valid, starts[pair_cls], 0)
    ge = jnp.where(valid, ends[pair_cls], 0)
    return (pair_blk.astype(jnp.int32), pair_cls.astype(jnp.int32),
            gs.astype(jnp.int32), ge.astype(jnp.int32))


def _grouped_matmul(x_sorted, Ws, bs, pair_blk, pair_cls, gs, ge, interpret=False):
    n = x_sorted.shape[0]
    nb = n // BLK
    g_total = nb + K - 1
    grid_spec = pltpu.PrefetchScalarGridSpec(
        num_scalar_prefetch=4,
        grid=(g_total,),
        in_specs=[
            pl.BlockSpec((BLK, D), lambda g, bid, cid, s, e: (bid[g], 0)),
            pl.BlockSpec((1, D, D), lambda g, bid, cid, s, e: (cid[g], 0, 0)),
            pl.BlockSpec((1, 1, D), lambda g, bid, cid, s, e: (cid[g], 0, 0)),
        ],
        out_specs=pl.BlockSpec((BLK, D), lambda g, bid, cid, s, e: (bid[g], 0)),
    )
    return pl.pallas_call(
        _gmm_body,
        grid_spec=grid_spec,
        out_shape=jax.ShapeDtypeStruct((n, D), jnp.float32),
        compiler_params=pltpu.CompilerParams(
            dimension_semantics=("arbitrary",)),
        interpret=interpret,
    )(pair_blk, pair_cls, gs, ge, x_sorted, Ws, bs.reshape(K, 1, D))


def kernel(x, Ws, bs):
    n = x.shape[0]
    cls = x[:, 0].astype(jnp.int32)
    feats = x[:, 1:]
    counts_all = _sc_counts(cls)
    totals = jnp.sum(counts_all, axis=0)
    class_start = jnp.cumsum(totals) - totals
    prefix_w = jnp.cumsum(counts_all, axis=0) - counts_all
    base = (class_start[None, :] + prefix_w).astype(jnp.int32)
    x_sorted, dest = _sc_route(cls, base, feats)
    pair_blk, pair_cls, gs, ge = _pair_metadata(totals[:K], n // BLK)
    y_sorted = _grouped_matmul(x_sorted, Ws, bs, pair_blk, pair_cls, gs, ge)
    return _sc_unsort(y_sorted, dest)
